# baseline (speedup 1.0000x reference)
import functools

import jax
import jax.numpy as jnp
from jax import lax
from jax.experimental import pallas as pl
from jax.experimental.pallas import tpu as pltpu
from jax.experimental.pallas import tpu_sc as plsc

_NC, _NS = 2, 16
_NW = _NC * _NS
_D = 256
_P = 64 * 1024
_PPW = _P // _NW
_CH = 64
_NCH = _PPW // _CH
_IDXROWS = _PPW // _CH
_NBUF = 3


@functools.partial(
    pl.kernel,
    mesh=plsc.VectorSubcoreMesh(core_axis_name="c", subcore_axis_name="s"),
    out_type=jax.ShapeDtypeStruct((_P, 2 * _D), jnp.float32),
    scratch_types=[
        pltpu.VMEM((_IDXROWS, _CH), jnp.int32),
        pltpu.VMEM((_IDXROWS, _CH), jnp.int32),
        pltpu.VMEM((_CH, 2 * _D), jnp.float32),
        pltpu.VMEM((_CH, 2 * _D), jnp.float32),
        pltpu.VMEM((_CH, 2 * _D), jnp.float32),
        pltpu.SemaphoreType.DMA,
        pltpu.SemaphoreType.DMA,
        pltpu.SemaphoreType.DMA,
        pltpu.SemaphoreType.DMA,
        pltpu.SemaphoreType.DMA,
        pltpu.SemaphoreType.DMA,
    ],
)
def _sc_lookup(idx_x_hbm, idx_y_hbm, col_hbm, row_hbm, out_hbm,
               idxx_v, idxy_v, buf0, buf1, buf2,
               sg0, sg1, sg2, so0, so1, so2):
    bufs = (buf0, buf1, buf2)
    sgs = (sg0, sg1, sg2)
    sos = (so0, so1, so2)
    wid = lax.axis_index("s") * _NC + lax.axis_index("c")
    base = wid * _PPW

    pltpu.sync_copy(idx_x_hbm.at[pl.ds(wid * _IDXROWS, _IDXROWS)], idxx_v)
    pltpu.sync_copy(idx_y_hbm.at[pl.ds(wid * _IDXROWS, _IDXROWS)], idxy_v)

    def start_gathers(c, b):
        pltpu.async_copy(
            col_hbm.at[idxx_v.at[c]], bufs[b].at[:, pl.ds(0, _D)], sgs[b])
        pltpu.async_copy(
            row_hbm.at[idxy_v.at[c]], bufs[b].at[:, pl.ds(_D, _D)], sgs[b])

    def wait_gathers(b):
        pltpu.make_async_copy(
            col_hbm.at[idxx_v.at[0]], bufs[b].at[:, pl.ds(0, _D)], sgs[b]).wait()
        pltpu.make_async_copy(
            row_hbm.at[idxy_v.at[0]], bufs[b].at[:, pl.ds(_D, _D)], sgs[b]).wait()

    def out_desc(c, b):
        return pltpu.make_async_copy(
            bufs[b], out_hbm.at[pl.ds(base + c * _CH, _CH)], sos[b])

    start_gathers(0, 0)
    start_gathers(1, 1)

    wait_gathers(0)
    out_desc(0, 0).start()
    start_gathers(2, 2)
    wait_gathers(1)
    out_desc(1, 1).start()
    out_desc(0, 0).wait()
    start_gathers(3, 0)
    wait_gathers(2)
    out_desc(2, 2).start()
    out_desc(1, 1).wait()
    start_gathers(4, 1)

    def trip(s, carry):
        for b in range(_NBUF):
            c = 3 * s + b
            wait_gathers(b)
            out_desc(c, b).start()
            bp = (b + 2) % _NBUF
            out_desc(c - 1, bp).wait()
            start_gathers(c + 2, bp)
        return carry

    lax.fori_loop(1, _NCH // 3, trip, 0)

    wait_gathers(0)
    out_desc(30, 0).start()
    out_desc(29, 2).wait()
    wait_gathers(1)
    out_desc(31, 1).start()
    out_desc(30, 0).wait()
    out_desc(31, 1).wait()


def kernel(position_inds, col_embed, row_embed):
    pi = position_inds.astype(jnp.int32)
    idx_x = pi[:, :, 0].reshape(_P // _CH, _CH)
    idx_y = pi[:, :, 1].reshape(_P // _CH, _CH)
    out = _sc_lookup(idx_x, idx_y, col_embed, row_embed)
    return out.reshape(64, 1024, 2 * _D)
